# baseline (speedup 1.0000x reference)
import functools

import jax
import jax.numpy as jnp
from jax import lax
from jax.experimental import pallas as pl
from jax.experimental.pallas import tpu as pltpu
from jax.experimental.pallas import tpu_sc as plsc

N = 10000
E = 320000
D = 128
NC = 2
NS = 16
C = 128
RPT = 624
TAIL0 = RPT * NS
TAIL = N - TAIL0
CHUNKS = 80
PAIRS = CHUNKS // 2
EPW = CHUNKS * C
EPAD = NC * NS * EPW


def _sc_aggregate(h_pad, src, dst3, zeros):
    mesh = plsc.VectorSubcoreMesh(core_axis_name="c", subcore_axis_name="s")

    @functools.partial(
        pl.kernel,
        mesh=mesh,
        out_type=jax.ShapeDtypeStruct((NC, N, D), jnp.float32),
        scratch_types=[
            [pltpu.VMEM((C,), jnp.int32) for _ in range(2)],
            [pltpu.VMEM((1, C), jnp.int32) for _ in range(2)],
            [pltpu.VMEM((C, D), jnp.float32) for _ in range(2)],
            pltpu.VMEM_SHARED((N, D), jnp.float32),
            [pltpu.SemaphoreType.DMA for _ in range(2)],
            [pltpu.SemaphoreType.DMA for _ in range(2)],
            [pltpu.SemaphoreType.DMA for _ in range(2)],
        ],
    )
    def agg_kernel(h_hbm, src_hbm, dst3_hbm, zeros_hbm, out_hbm,
                   src_v, dst_v, rows_v, acc_sh, sems_s, sems_d, sems_g):
        c = lax.axis_index("c")
        s = lax.axis_index("s")
        w = c * NS + s
        row0 = s * RPT

        @pl.when(c == 0)
        def _():
            pltpu.sync_copy(h_hbm.at[pl.ds(row0, RPT)],
                            acc_sh.at[pl.ds(row0, RPT)])

            @pl.when(s == NS - 1)
            def _():
                pltpu.sync_copy(h_hbm.at[pl.ds(TAIL0, TAIL)],
                                acc_sh.at[pl.ds(TAIL0, TAIL)])

        @pl.when(c != 0)
        def _():
            pltpu.sync_copy(zeros_hbm.at[pl.ds(row0, RPT)],
                            acc_sh.at[pl.ds(row0, RPT)])

            @pl.when(s == NS - 1)
            def _():
                pltpu.sync_copy(zeros_hbm.at[pl.ds(TAIL0, TAIL)],
                                acc_sh.at[pl.ds(TAIL0, TAIL)])

        plsc.subcore_barrier()

        def body(g, carry):
            r = w * CHUNKS + g
            eb = pl.multiple_of(r * C, 8)
            pltpu.sync_copy(src_hbm.at[pl.ds(eb, C)], src_v[0])
            pltpu.sync_copy(dst3_hbm.at[r], dst_v[0])
            pltpu.async_copy(h_hbm.at[src_v[0]], rows_v[0], sems_g[0]).wait()
            pltpu.sync_copy(rows_v[0], acc_sh.at[dst_v[0].at[0]], add=True)
            return carry

        lax.fori_loop(0, CHUNKS, body, 0)

        plsc.subcore_barrier()
        pltpu.sync_copy(acc_sh.at[pl.ds(row0, RPT)],
                        out_hbm.at[c, pl.ds(row0, RPT)])

        @pl.when(s == NS - 1)
        def _():
            pltpu.sync_copy(acc_sh.at[pl.ds(TAIL0, TAIL)],
                            out_hbm.at[c, pl.ds(TAIL0, TAIL)])

    return agg_kernel(h_pad, src, dst3, zeros)


def _tc_mlp(agg, W, b):
    d_out = W.shape[1]
    BR = 1000

    def mm_kernel(a_ref, w_ref, b_ref, o_ref):
        x = a_ref[0] + a_ref[1]
        o_ref[...] = jnp.dot(x, w_ref[...],
                             preferred_element_type=jnp.float32) + b_ref[...]

    return pl.pallas_call(
        mm_kernel,
        grid=(N // BR,),
        in_specs=[
            pl.BlockSpec((2, BR, D), lambda i: (0, i, 0)),
            pl.BlockSpec((D, d_out), lambda i: (0, 0)),
            pl.BlockSpec((1, d_out), lambda i: (0, 0)),
        ],
        out_specs=pl.BlockSpec((BR, d_out), lambda i: (i, 0)),
        out_shape=jax.ShapeDtypeStruct((N, d_out), jnp.float32),
    )(agg, W, b.reshape(1, d_out))


def kernel(features, edge_index, W_in, b_in, W_hid, b_hid, W_out, b_out):
    pad = EPAD - E
    src = jnp.concatenate([edge_index[0],
                           jnp.full((pad,), N, jnp.int32)])
    dst3 = jnp.concatenate([edge_index[1],
                            jnp.zeros((pad,), jnp.int32)])
    dst3 = dst3.reshape(EPAD // C, 1, C)
    zeros = jnp.zeros((N, D), jnp.float32)
    h = features
    for W, b in ((W_in, b_in), (W_hid, b_hid), (W_out, b_out)):
        h_pad = jnp.concatenate([h, jnp.zeros((8, D), jnp.float32)])
        agg = _sc_aggregate(h_pad, src, dst3, zeros)
        h = _tc_mlp(agg, W, b)
    return h


# speedup vs baseline: 2.2081x; 2.2081x over previous
import functools

import jax
import jax.numpy as jnp
from jax import lax
from jax.experimental import pallas as pl
from jax.experimental.pallas import tpu as pltpu
from jax.experimental.pallas import tpu_sc as plsc

N = 10000
E = 320000
D = 128
NC = 2
NS = 16
C = 128
RPT = 624
TAIL0 = RPT * NS
TAIL = N - TAIL0
CHUNKS = 80
PAIRS = CHUNKS // 2
EPW = CHUNKS * C
EPAD = NC * NS * EPW


def _sc_aggregate(h_pad, src, dst3, zeros):
    mesh = plsc.VectorSubcoreMesh(core_axis_name="c", subcore_axis_name="s")

    @functools.partial(
        pl.kernel,
        mesh=mesh,
        out_type=jax.ShapeDtypeStruct((NC, N, D), jnp.float32),
        scratch_types=[
            [pltpu.VMEM((C,), jnp.int32) for _ in range(2)],
            [pltpu.VMEM((1, C), jnp.int32) for _ in range(2)],
            [pltpu.VMEM((C, D), jnp.float32) for _ in range(2)],
            pltpu.VMEM_SHARED((N, D), jnp.float32),
            [pltpu.SemaphoreType.DMA for _ in range(2)],
            [pltpu.SemaphoreType.DMA for _ in range(2)],
            [pltpu.SemaphoreType.DMA for _ in range(2)],
        ],
    )
    def agg_kernel(h_hbm, src_hbm, dst3_hbm, zeros_hbm, out_hbm,
                   src_v, dst_v, rows_v, acc_sh, sems_s, sems_d, sems_g):
        c = lax.axis_index("c")
        s = lax.axis_index("s")
        w = c * NS + s
        row0 = s * RPT

        @pl.when(c == 0)
        def _():
            pltpu.sync_copy(h_hbm.at[pl.ds(row0, RPT)],
                            acc_sh.at[pl.ds(row0, RPT)])

            @pl.when(s == NS - 1)
            def _():
                pltpu.sync_copy(h_hbm.at[pl.ds(TAIL0, TAIL)],
                                acc_sh.at[pl.ds(TAIL0, TAIL)])

        @pl.when(c != 0)
        def _():
            pltpu.sync_copy(zeros_hbm.at[pl.ds(row0, RPT)],
                            acc_sh.at[pl.ds(row0, RPT)])

            @pl.when(s == NS - 1)
            def _():
                pltpu.sync_copy(zeros_hbm.at[pl.ds(TAIL0, TAIL)],
                                acc_sh.at[pl.ds(TAIL0, TAIL)])

        plsc.subcore_barrier()

        def body(g, carry):
            r = w * CHUNKS + g
            eb = pl.multiple_of(r * C, 8)
            pltpu.sync_copy(src_hbm.at[pl.ds(eb, C)], src_v[0])
            pltpu.sync_copy(dst3_hbm.at[r], dst_v[0])
            pltpu.async_copy(h_hbm.at[src_v[0]], rows_v[0], sems_g[0]).wait()
            pltpu.sync_copy(rows_v[0], acc_sh.at[dst_v[0].at[0]], add=True)
            return carry

        lax.fori_loop(0, CHUNKS, body, 0)

        plsc.subcore_barrier()
        pltpu.sync_copy(acc_sh.at[pl.ds(row0, RPT)],
                        out_hbm.at[c, pl.ds(row0, RPT)])

        @pl.when(s == NS - 1)
        def _():
            pltpu.sync_copy(acc_sh.at[pl.ds(TAIL0, TAIL)],
                            out_hbm.at[c, pl.ds(TAIL0, TAIL)])

    return agg_kernel(h_pad, src, dst3, zeros)


def _tc_mlp(agg, W, b):
    d_out = W.shape[1]
    BR = 1000

    def mm_kernel(a_ref, w_ref, b_ref, o_ref):
        x = a_ref[0] + a_ref[1]
        o_ref[...] = jnp.dot(x, w_ref[...],
                             preferred_element_type=jnp.float32) + b_ref[...]

    return pl.pallas_call(
        mm_kernel,
        grid=(N // BR,),
        in_specs=[
            pl.BlockSpec((2, BR, D), lambda i: (0, i, 0)),
            pl.BlockSpec((D, d_out), lambda i: (0, 0)),
            pl.BlockSpec((1, d_out), lambda i: (0, 0)),
        ],
        out_specs=pl.BlockSpec((BR, d_out), lambda i: (i, 0)),
        out_shape=jax.ShapeDtypeStruct((N, d_out), jnp.float32),
    )(agg, W, b.reshape(1, d_out))


def kernel(features, edge_index, W_in, b_in, W_hid, b_hid, W_out, b_out):
    pad = EPAD - E
    pad_iota = jnp.arange(pad, dtype=jnp.int32)
    src = jnp.concatenate([edge_index[0], N + (pad_iota % 8)])
    dst3 = jnp.concatenate([edge_index[1], pad_iota % N])
    dst3 = dst3.reshape(EPAD // C, 1, C)
    zeros = jnp.zeros((N, D), jnp.float32)
    h = features
    for W, b in ((W_in, b_in), (W_hid, b_hid), (W_out, b_out)):
        h_pad = jnp.concatenate([h, jnp.zeros((8, D), jnp.float32)])
        agg = _sc_aggregate(h_pad, src, dst3, zeros)
        h = _tc_mlp(agg, W, b)
    return h


# speedup vs baseline: 2.9597x; 1.3404x over previous
import functools

import jax
import jax.numpy as jnp
from jax import lax
from jax.experimental import pallas as pl
from jax.experimental.pallas import tpu as pltpu
from jax.experimental.pallas import tpu_sc as plsc

N = 10000
E = 320000
D = 128
NC = 2
NS = 16
C = 128
RPT = 624
TAIL0 = RPT * NS
TAIL = N - TAIL0
CHUNKS = 80
PAIRS = CHUNKS // 2
EPW = CHUNKS * C
EPAD = NC * NS * EPW


def _sc_aggregate(h_pad, src, dst3, zeros):
    mesh = plsc.VectorSubcoreMesh(core_axis_name="c", subcore_axis_name="s")

    @functools.partial(
        pl.kernel,
        mesh=mesh,
        out_type=jax.ShapeDtypeStruct((NC, N, D), jnp.float32),
        scratch_types=[
            [pltpu.VMEM((C,), jnp.int32) for _ in range(2)],
            [pltpu.VMEM((1, C), jnp.int32) for _ in range(2)],
            [pltpu.VMEM((C, D), jnp.float32) for _ in range(2)],
            pltpu.VMEM_SHARED((N, D), jnp.float32),
            [pltpu.SemaphoreType.DMA for _ in range(2)],
            [pltpu.SemaphoreType.DMA for _ in range(2)],
            [pltpu.SemaphoreType.DMA for _ in range(2)],
        ],
    )
    def agg_kernel(h_hbm, src_hbm, dst3_hbm, zeros_hbm, out_hbm,
                   src_v, dst_v, rows_v, acc_sh, sems_s, sems_d, sems_g):
        c = lax.axis_index("c")
        s = lax.axis_index("s")
        w = c * NS + s
        row0 = s * RPT

        @pl.when(c == 0)
        def _():
            pltpu.sync_copy(h_hbm.at[pl.ds(row0, RPT)],
                            acc_sh.at[pl.ds(row0, RPT)])

            @pl.when(s == NS - 1)
            def _():
                pltpu.sync_copy(h_hbm.at[pl.ds(TAIL0, TAIL)],
                                acc_sh.at[pl.ds(TAIL0, TAIL)])

        @pl.when(c != 0)
        def _():
            pltpu.sync_copy(zeros_hbm.at[pl.ds(row0, RPT)],
                            acc_sh.at[pl.ds(row0, RPT)])

            @pl.when(s == NS - 1)
            def _():
                pltpu.sync_copy(zeros_hbm.at[pl.ds(TAIL0, TAIL)],
                                acc_sh.at[pl.ds(TAIL0, TAIL)])

        plsc.subcore_barrier()

        def body(p, carry):
            r0 = w * CHUNKS + p * 2
            cp_s, cp_d, cp_g = [None, None], [None, None], [None, None]
            for j in range(2):
                eb = pl.multiple_of((r0 + j) * C, 8)
                cp_s[j] = pltpu.async_copy(src_hbm.at[pl.ds(eb, C)],
                                           src_v[j], sems_s[j])
                cp_d[j] = pltpu.async_copy(dst3_hbm.at[r0 + j],
                                           dst_v[j], sems_d[j])
            for j in range(2):
                cp_s[j].wait()
                cp_g[j] = pltpu.async_copy(h_hbm.at[src_v[j]],
                                           rows_v[j], sems_g[j])
            for j in range(2):
                cp_g[j].wait()
                cp_d[j].wait()
                pltpu.sync_copy(rows_v[j], acc_sh.at[dst_v[j].at[0]],
                                add=True)
            return carry

        lax.fori_loop(0, PAIRS, body, 0)

        plsc.subcore_barrier()
        pltpu.sync_copy(acc_sh.at[pl.ds(row0, RPT)],
                        out_hbm.at[c, pl.ds(row0, RPT)])

        @pl.when(s == NS - 1)
        def _():
            pltpu.sync_copy(acc_sh.at[pl.ds(TAIL0, TAIL)],
                            out_hbm.at[c, pl.ds(TAIL0, TAIL)])

    return agg_kernel(h_pad, src, dst3, zeros)


def _tc_mlp(agg, W, b):
    d_out = W.shape[1]
    BR = 1000

    def mm_kernel(a_ref, w_ref, b_ref, o_ref):
        x = a_ref[0] + a_ref[1]
        o_ref[...] = jnp.dot(x, w_ref[...],
                             preferred_element_type=jnp.float32) + b_ref[...]

    return pl.pallas_call(
        mm_kernel,
        grid=(N // BR,),
        in_specs=[
            pl.BlockSpec((2, BR, D), lambda i: (0, i, 0)),
            pl.BlockSpec((D, d_out), lambda i: (0, 0)),
            pl.BlockSpec((1, d_out), lambda i: (0, 0)),
        ],
        out_specs=pl.BlockSpec((BR, d_out), lambda i: (i, 0)),
        out_shape=jax.ShapeDtypeStruct((N, d_out), jnp.float32),
    )(agg, W, b.reshape(1, d_out))


def kernel(features, edge_index, W_in, b_in, W_hid, b_hid, W_out, b_out):
    pad = EPAD - E
    pad_iota = jnp.arange(pad, dtype=jnp.int32)
    src = jnp.concatenate([edge_index[0], N + (pad_iota % 8)])
    dst3 = jnp.concatenate([edge_index[1], pad_iota % N])
    dst3 = dst3.reshape(EPAD // C, 1, C)
    zeros = jnp.zeros((N, D), jnp.float32)
    h = features
    for W, b in ((W_in, b_in), (W_hid, b_hid), (W_out, b_out)):
        h_pad = jnp.concatenate([h, jnp.zeros((8, D), jnp.float32)])
        agg = _sc_aggregate(h_pad, src, dst3, zeros)
        h = _tc_mlp(agg, W, b)
    return h
